# Initial kernel scaffold; baseline (speedup 1.0000x reference)
import jax
import jax.numpy as jnp
from jax.experimental import pallas as pl


def kernel(x, edge_index, batch, cell_type_batch, W, b):
    raise NotImplementedError("write your pallas kernel here")


# speedup vs baseline: 17.9655x; 17.9655x over previous
import functools

import jax
import jax.numpy as jnp
from jax import lax
from jax.experimental import pallas as pl
from jax.experimental.pallas import tpu as pltpu
from jax.experimental.pallas import tpu_sc as plsc

N = 10000
E = 160000
D = 256
HALF = 128
T = 100
K = 125
EROWS = E // K
NC = 2
NS = 16
ROWS_PER_TILE = EROWS // NS
ROWS_PER_WORKER = EROWS // (NC * NS)
NSEG = 624
NTAIL = N - NS * NSEG
DEG_W = 16

_MESH = plsc.VectorSubcoreMesh(core_axis_name="c", subcore_axis_name="s")


NW = NC * NS
EPW = E // NW
EPW_PAD = 5008


def _deg_body(dst_hbm, zeros_hbm, out_hbm, idx_v, cnt_v):
    c = lax.axis_index("c")
    s = lax.axis_index("s")
    wid = c * NS + s
    pltpu.sync_copy(zeros_hbm, cnt_v)
    pltpu.sync_copy(dst_hbm.at[wid].at[0], idx_v)
    ones16 = jnp.ones((16,), jnp.float32)
    lanes = lax.broadcasted_iota(jnp.int32, (16,), 0)

    @pl.loop(0, EPW_PAD // 16)
    def _(j):
        base = j * 16
        iv = plsc.load_gather(idx_v, [lanes + base])
        mask = (lanes + base) < EPW
        plsc.addupdate_scatter(cnt_v, [iv], ones16, mask=mask)

    pltpu.sync_copy(cnt_v, out_hbm.at[wid].at[0])


_deg_call = pl.kernel(
    _deg_body,
    out_type=jax.ShapeDtypeStruct((NW, 1, N), jnp.float32),
    mesh=_MESH,
    scratch_types=[
        pltpu.VMEM((EPW_PAD,), jnp.int32),
        pltpu.VMEM((N,), jnp.float32),
    ],
    compiler_params=pltpu.CompilerParams(needs_layout_passes=False),
)


def _scatter_body(g_hbm, src_hbm, dst_hbm, acc_hbm, srci_v, dsti_v, buf_v,
                  acc_sh):
    c = lax.axis_index("c")
    s = lax.axis_index("s")
    seg = pl.ds(s * NSEG, NSEG)
    tail = pl.ds(NS * NSEG, NTAIL)
    erows = pl.ds(s * ROWS_PER_TILE, ROWS_PER_TILE)
    pltpu.sync_copy(src_hbm.at[erows], srci_v)
    pltpu.sync_copy(dst_hbm.at[erows], dsti_v)
    for cc in range(NC):
        @pl.when(c == cc)
        def _(cc=cc):
            pltpu.sync_copy(g_hbm.at[cc].at[seg], acc_sh.at[seg])

            @pl.when(s == NS - 1)
            def _():
                pltpu.sync_copy(g_hbm.at[cc].at[tail], acc_sh.at[tail])

    plsc.subcore_barrier()
    for cc in range(NC):
        @pl.when(c == cc)
        def _(cc=cc):
            @pl.loop(0, ROWS_PER_TILE)
            def _(j):
                pltpu.sync_copy(g_hbm.at[cc].at[srci_v.at[j]], buf_v)
                pltpu.sync_copy(buf_v, acc_sh.at[dsti_v.at[j]], add=True)

    plsc.subcore_barrier()
    for cc in range(NC):
        @pl.when(c == cc)
        def _(cc=cc):
            pltpu.sync_copy(acc_sh.at[seg], acc_hbm.at[cc].at[seg])

            @pl.when(s == NS - 1)
            def _():
                pltpu.sync_copy(acc_sh.at[tail], acc_hbm.at[cc].at[tail])


_scatter_call = pl.kernel(
    _scatter_body,
    out_type=jax.ShapeDtypeStruct((NC, N, HALF), jnp.float32),
    mesh=_MESH,
    scratch_types=[
        pltpu.VMEM((ROWS_PER_TILE, K), jnp.int32),
        pltpu.VMEM((ROWS_PER_TILE, K), jnp.int32),
        pltpu.VMEM((K, HALF), jnp.float32),
        pltpu.VMEM_SHARED((N, HALF), jnp.float32),
    ],
)


_RB = 2000


def _mm_body(x_ref, w_ref, dinv_ref, o_ref):
    acc = jnp.dot(x_ref[...], w_ref[...], preferred_element_type=jnp.float32)
    o_ref[0] = acc * dinv_ref[...]


_mm_call = pl.pallas_call(
    _mm_body,
    grid=(N // _RB, NC),
    in_specs=[
        pl.BlockSpec((_RB, D), lambda i, c: (i, 0)),
        pl.BlockSpec((D, HALF), lambda i, c: (0, c)),
        pl.BlockSpec((_RB, 1), lambda i, c: (i, 0)),
    ],
    out_specs=pl.BlockSpec((1, _RB, HALF), lambda i, c: (c, i, 0)),
    out_shape=jax.ShapeDtypeStruct((NC, N, HALF), jnp.float32),
)


def _pool_body(acc_ref, dinv_ref, b_ref, ct_ref, o_ref, cnt_ref):
    i = pl.program_id(0)

    @pl.when(i == 0)
    def _():
        o_ref[...] = jnp.zeros_like(o_ref)
        cnt_ref[...] = jnp.zeros_like(cnt_ref)

    rows = jnp.concatenate([acc_ref[0], acc_ref[1]], axis=1)
    h = jnp.maximum(rows * dinv_ref[...] + b_ref[...], 0.0)
    onehot = (lax.broadcasted_iota(jnp.int32, (T, _RB), 0)
              == ct_ref[0]).astype(jnp.float32)
    o_ref[...] += jnp.dot(onehot, h, preferred_element_type=jnp.float32)
    cnt_ref[...] += jnp.sum(onehot, axis=1, keepdims=True)

    @pl.when(i == pl.num_programs(0) - 1)
    def _():
        o_ref[...] = o_ref[...] / jnp.maximum(cnt_ref[...], 1.0)


_pool_call = pl.pallas_call(
    _pool_body,
    grid=(N // _RB,),
    in_specs=[
        pl.BlockSpec((NC, _RB, HALF), lambda i: (0, i, 0)),
        pl.BlockSpec((_RB, 1), lambda i: (i, 0)),
        pl.BlockSpec((1, D), lambda i: (0, 0)),
        pl.BlockSpec((1, 1, _RB), lambda i: (i, 0, 0)),
    ],
    out_specs=pl.BlockSpec((T, D), lambda i: (0, 0)),
    out_shape=jax.ShapeDtypeStruct((T, D), jnp.float32),
    scratch_shapes=[pltpu.VMEM((T, 1), jnp.float32)],
)


def kernel(x, edge_index, batch, cell_type_batch, W, b):
    src = edge_index[0].reshape(EROWS, K)
    dst = edge_index[1].reshape(EROWS, K)
    dstp = jnp.pad(edge_index[1].reshape(NW, EPW),
                   ((0, 0), (0, EPW_PAD - EPW)))[:, None, :]
    degp = _deg_call(dstp, jnp.zeros((N,), jnp.float32))
    deg = jnp.sum(degp[:, 0, :], axis=0) + 1.0
    dinv = lax.rsqrt(deg).reshape(N, 1)
    g2 = _mm_call(x, W, dinv)
    acc2 = _scatter_call(g2, src, dst)
    return _pool_call(acc2, dinv, b.reshape(1, D),
                      cell_type_batch.reshape(N // _RB, 1, _RB))


# speedup vs baseline: 21.9681x; 1.2228x over previous
import functools

import jax
import jax.numpy as jnp
from jax import lax
from jax.experimental import pallas as pl
from jax.experimental.pallas import tpu as pltpu
from jax.experimental.pallas import tpu_sc as plsc

N = 10000
E = 160000
D = 256
HALF = 128
T = 100
K = 125
EROWS = E // K
NC = 2
NS = 16
ROWS_PER_TILE = EROWS // NS
PH_ROWS = 40
ROWS_PER_WORKER = EROWS // (NC * NS)
NSEG = 624
NTAIL = N - NS * NSEG
DEG_W = 16

_MESH = plsc.VectorSubcoreMesh(core_axis_name="c", subcore_axis_name="s")


NW = NC * NS
EPW = E // NW
EPW_PAD = 5008


def _deg_body(dst_hbm, zeros_hbm, out_hbm, idx_v, cnt_v):
    c = lax.axis_index("c")
    s = lax.axis_index("s")
    wid = c * NS + s
    pltpu.sync_copy(zeros_hbm, cnt_v)
    pltpu.sync_copy(dst_hbm.at[wid].at[0], idx_v)
    ones16 = jnp.ones((16,), jnp.float32)
    lanes = lax.broadcasted_iota(jnp.int32, (16,), 0)

    @pl.loop(0, EPW_PAD // 16)
    def _(j):
        base = j * 16
        iv = plsc.load_gather(idx_v, [lanes + base])
        mask = (lanes + base) < EPW
        plsc.addupdate_scatter(cnt_v, [iv], ones16, mask=mask)

    pltpu.sync_copy(cnt_v, out_hbm.at[wid].at[0])


_deg_call = pl.kernel(
    _deg_body,
    out_type=jax.ShapeDtypeStruct((NW, 1, N), jnp.float32),
    mesh=_MESH,
    scratch_types=[
        pltpu.VMEM((EPW_PAD,), jnp.int32),
        pltpu.VMEM((N,), jnp.float32),
    ],
    compiler_params=pltpu.CompilerParams(needs_layout_passes=False),
)


def _scatter_body(g_hbm, src_hbm, dst_hbm, acc_hbm, srci_v, dsti_v, buf0_v,
                  buf1_v, sem0, sem1, acc_sh):
    c = lax.axis_index("c")
    s = lax.axis_index("s")
    seg = pl.ds(s * NSEG, NSEG)
    tail = pl.ds(NS * NSEG, NTAIL)
    for cc in range(NC):
        @pl.when(c == cc)
        def _(cc=cc):
            pltpu.sync_copy(g_hbm.at[cc].at[seg], acc_sh.at[seg])

            @pl.when(s == NS - 1)
            def _():
                pltpu.sync_copy(g_hbm.at[cc].at[tail], acc_sh.at[tail])

    plsc.subcore_barrier()
    bufs = (buf0_v, buf1_v)
    sems = (sem0, sem1)
    for cc in range(NC):
        @pl.when(c == cc)
        def _(cc=cc):
            g = g_hbm.at[cc]
            for ph in range(ROWS_PER_TILE // PH_ROWS):
                base = s * ROWS_PER_TILE + ph * PH_ROWS
                pltpu.sync_copy(src_hbm.at[pl.ds(base, PH_ROWS)], srci_v)
                pltpu.sync_copy(dst_hbm.at[pl.ds(base, PH_ROWS)], dsti_v)
                pltpu.async_copy(g.at[srci_v.at[0]], bufs[0], sems[0])

                @pl.loop(0, PH_ROWS, step=2)
                def _(t):
                    for b2 in range(2):
                        j = t + b2
                        buf, sem = bufs[b2], sems[b2]
                        obuf, osem = bufs[1 - b2], sems[1 - b2]
                        pltpu.make_async_copy(g.at[srci_v.at[0]], buf,
                                              sem).wait()
                        nj = j + 1

                        @pl.when(nj < PH_ROWS)
                        def _():
                            pltpu.async_copy(g.at[srci_v.at[nj]], obuf, osem)

                        pltpu.sync_copy(buf, acc_sh.at[dsti_v.at[j]],
                                        add=True)

    plsc.subcore_barrier()
    for cc in range(NC):
        @pl.when(c == cc)
        def _(cc=cc):
            pltpu.sync_copy(acc_sh.at[seg], acc_hbm.at[cc].at[seg])

            @pl.when(s == NS - 1)
            def _():
                pltpu.sync_copy(acc_sh.at[tail], acc_hbm.at[cc].at[tail])


_scatter_call = pl.kernel(
    _scatter_body,
    out_type=jax.ShapeDtypeStruct((NC, N, HALF), jnp.float32),
    mesh=_MESH,
    scratch_types=[
        pltpu.VMEM((PH_ROWS, K), jnp.int32),
        pltpu.VMEM((PH_ROWS, K), jnp.int32),
        pltpu.VMEM((K, HALF), jnp.float32),
        pltpu.VMEM((K, HALF), jnp.float32),
        pltpu.SemaphoreType.DMA,
        pltpu.SemaphoreType.DMA,
        pltpu.VMEM_SHARED((N, HALF), jnp.float32),
    ],
)


_RB = 2000


def _mm_body(x_ref, w_ref, dinv_ref, o_ref):
    acc = jnp.dot(x_ref[...], w_ref[...], preferred_element_type=jnp.float32)
    o_ref[0] = acc * dinv_ref[...]


_mm_call = pl.pallas_call(
    _mm_body,
    grid=(N // _RB, NC),
    in_specs=[
        pl.BlockSpec((_RB, D), lambda i, c: (i, 0)),
        pl.BlockSpec((D, HALF), lambda i, c: (0, c)),
        pl.BlockSpec((_RB, 1), lambda i, c: (i, 0)),
    ],
    out_specs=pl.BlockSpec((1, _RB, HALF), lambda i, c: (c, i, 0)),
    out_shape=jax.ShapeDtypeStruct((NC, N, HALF), jnp.float32),
)


def _pool_body(acc_ref, dinv_ref, b_ref, ct_ref, o_ref, cnt_ref):
    i = pl.program_id(0)

    @pl.when(i == 0)
    def _():
        o_ref[...] = jnp.zeros_like(o_ref)
        cnt_ref[...] = jnp.zeros_like(cnt_ref)

    rows = jnp.concatenate([acc_ref[0], acc_ref[1]], axis=1)
    h = jnp.maximum(rows * dinv_ref[...] + b_ref[...], 0.0)
    onehot = (lax.broadcasted_iota(jnp.int32, (T, _RB), 0)
              == ct_ref[0]).astype(jnp.float32)
    o_ref[...] += jnp.dot(onehot, h, preferred_element_type=jnp.float32)
    cnt_ref[...] += jnp.sum(onehot, axis=1, keepdims=True)

    @pl.when(i == pl.num_programs(0) - 1)
    def _():
        o_ref[...] = o_ref[...] / jnp.maximum(cnt_ref[...], 1.0)


_pool_call = pl.pallas_call(
    _pool_body,
    grid=(N // _RB,),
    in_specs=[
        pl.BlockSpec((NC, _RB, HALF), lambda i: (0, i, 0)),
        pl.BlockSpec((_RB, 1), lambda i: (i, 0)),
        pl.BlockSpec((1, D), lambda i: (0, 0)),
        pl.BlockSpec((1, 1, _RB), lambda i: (i, 0, 0)),
    ],
    out_specs=pl.BlockSpec((T, D), lambda i: (0, 0)),
    out_shape=jax.ShapeDtypeStruct((T, D), jnp.float32),
    scratch_shapes=[pltpu.VMEM((T, 1), jnp.float32)],
)


def kernel(x, edge_index, batch, cell_type_batch, W, b):
    src = edge_index[0].reshape(EROWS, K)
    dst = edge_index[1].reshape(EROWS, K)
    dstp = jnp.pad(edge_index[1].reshape(NW, EPW),
                   ((0, 0), (0, EPW_PAD - EPW)))[:, None, :]
    degp = _deg_call(dstp, jnp.zeros((N,), jnp.float32))
    deg = jnp.sum(degp[:, 0, :], axis=0) + 1.0
    dinv = lax.rsqrt(deg).reshape(N, 1)
    g2 = _mm_call(x, W, dinv)
    acc2 = _scatter_call(g2, src, dst)
    return _pool_call(acc2, dinv, b.reshape(1, D),
                      cell_type_batch.reshape(N // _RB, 1, _RB))
